# baseline (speedup 1.0000x reference)
from functools import partial

import jax
import jax.numpy as jnp
from jax.experimental import pallas as pl
from jax.experimental.pallas import tpu as pltpu

ENERGY_MOMENTUM = 0.9
BASE_PROB = 0.1
ENERGY_SCALE = 2.0
LAYER_IDX = 0

_CHUNK = 512


def _partials_kernel(x_ref, part_ref):
    x = x_ref[...]
    s = jnp.sum(x, axis=(0, 2))
    sq = jnp.sum(x * x, axis=(0, 2))
    part_ref[0, 0, :] = s
    part_ref[0, 1, :] = sq


def _epilogue_kernel(part_ref, he_ref, probs_ref, *, n_total):
    part = part_ref[...]
    ssum = jnp.sum(part[:, 0, :], axis=0)
    ssq = jnp.sum(part[:, 1, :], axis=0)
    n = jnp.float32(n_total)
    head_var = (ssq - ssum * ssum / n) / (n - 1.0)
    mx = jnp.max(head_var)
    head_var = jnp.where(mx > 0, head_var / (mx + 1e-08), head_var)

    he = he_ref[...]
    new_row = ENERGY_MOMENTUM * he[LAYER_IDX, :] + (1.0 - ENERGY_MOMENTUM) * head_var
    row_ids = jax.lax.broadcasted_iota(jnp.int32, he.shape, 0)
    new_energy = jnp.where(row_ids == LAYER_IDX, new_row[None, :], he)

    inv = 1.0 / (new_energy + 0.1)
    inv = inv / jnp.max(inv)
    probs = BASE_PROB * (1.0 + ENERGY_SCALE * inv)
    probs_ref[...] = jnp.clip(probs, 0.0, 1.0)


def kernel(output, head_energy):
    B, T, H, D = output.shape
    x = output.reshape(B * T, H, D)
    rows = B * T
    n_steps = rows // _CHUNK
    n_total = rows * D

    partials = pl.pallas_call(
        _partials_kernel,
        grid=(n_steps,),
        in_specs=[pl.BlockSpec((_CHUNK, H, D), lambda i: (i, 0, 0))],
        out_specs=pl.BlockSpec((1, 8, H), lambda i: (i, 0, 0)),
        out_shape=jax.ShapeDtypeStruct((n_steps, 8, H), jnp.float32),
        compiler_params=pltpu.CompilerParams(
            dimension_semantics=("parallel",),
        ),
    )(x)

    return pl.pallas_call(
        partial(_epilogue_kernel, n_total=n_total),
        in_specs=[
            pl.BlockSpec(partials.shape, lambda: (0, 0, 0)),
            pl.BlockSpec(head_energy.shape, lambda: (0, 0)),
        ],
        out_specs=pl.BlockSpec(head_energy.shape, lambda: (0, 0)),
        out_shape=jax.ShapeDtypeStruct(head_energy.shape, jnp.float32),
    )(partials, head_energy)
